# baseline (speedup 1.0000x reference)
import functools

import jax
import jax.numpy as jnp
from jax import lax
from jax.experimental import pallas as pl
from jax.experimental.pallas import tpu as pltpu
from jax.experimental.pallas import tpu_sc as plsc

T = 4096
D = 768
E = 64
K = 2
F = 1024
CAP = 256
TRASH = E * CAP
NW = 32
TPW = T // NW
TCHUNK = 128
D2 = D // 2


def _pack_row(v):
    bits = jax.lax.bitcast_convert_type(v, jnp.int32) + 0x8000
    hi16 = jax.lax.shift_right_logical(bits, 16)
    lo = hi16[:, :D2] & 0xFFFF
    hi = jax.lax.shift_left(hi16[:, D2:], 16)
    return lo | hi


def _unpack_row(w):
    a = jax.lax.bitcast_convert_type(jax.lax.shift_left(w, 16), jnp.float32)
    b = jax.lax.bitcast_convert_type(w & jnp.int32(-65536), jnp.float32)
    return jnp.concatenate([a, b], axis=1)


_RTB = 512


def _router_body(x_ref, wr_ref, si0_ref, si1_ref, gi0_ref, gi1_ref, w_ref,
                 xpk_ref, off_ref):
    @pl.when(pl.program_id(0) == 0)
    def _init():
        off_ref[...] = jnp.zeros((1, E), jnp.float32)

    x = x_ref[...]
    xpk_ref[...] = _pack_row(x)
    logits = jnp.dot(x, wr_ref[...], preferred_element_type=jnp.float32)
    lanes = lax.broadcasted_iota(jnp.int32, logits.shape, 1)
    m0 = jnp.max(logits, axis=-1, keepdims=True)
    i0 = jnp.min(jnp.where(logits == m0, lanes, E), axis=-1, keepdims=True)
    neg = jnp.float32(-jnp.inf)
    p1 = jnp.where(lanes == i0, neg, logits)
    m1 = jnp.max(p1, axis=-1, keepdims=True)
    i1 = jnp.min(jnp.where(p1 == m1, lanes, E), axis=-1, keepdims=True)
    oh0 = (lanes == i0).astype(jnp.float32)
    oh1 = (lanes == i1).astype(jnp.float32)
    h = oh0 + oh1
    _B = 256
    rr = lax.broadcasted_iota(jnp.int32, (_B, _B), 0)
    cc = lax.broadcasted_iota(jnp.int32, (_B, _B), 1)
    tri = (rr > cc).astype(jnp.float32)
    parts = []
    off = off_ref[...]
    for b in range(_RTB // _B):
        hb = h[b * _B:(b + 1) * _B, :]
        parts.append(
            jnp.dot(tri, hb, preferred_element_type=jnp.float32) + off)
        off = off + jnp.sum(hb, axis=0, keepdims=True)
    off_ref[...] = off
    cex = jnp.concatenate(parts, axis=0)
    pos0 = jnp.sum(cex * oh0, axis=-1, keepdims=True).astype(jnp.int32)
    pos1 = jnp.sum(cex * oh1, axis=-1, keepdims=True).astype(jnp.int32)
    q = jnp.exp(m1 - m0)
    w1 = q / (1.0 + q)
    w0 = 1.0 - w1
    v0 = pos0 < CAP
    v1 = pos1 < CAP
    slot0 = i0 * CAP + jnp.minimum(pos0, CAP - 1)
    slot1 = i1 * CAP + jnp.minimum(pos1, CAP - 1)
    si0_ref[...] = jnp.where(v0, slot0, TRASH)
    si1_ref[...] = jnp.where(v1, slot1, TRASH)
    gi0_ref[...] = slot0
    gi1_ref[...] = slot1
    w_ref[...] = jnp.concatenate(
        [jnp.where(v0, w0, 0.0), jnp.where(v1, w1, 0.0)], axis=1)


def _router(x, w_router):
    return pl.pallas_call(
        _router_body,
        grid=(T // _RTB,),
        in_specs=[
            pl.BlockSpec((_RTB, D), lambda i: (i, 0)),
            pl.BlockSpec((D, E), lambda i: (0, 0)),
        ],
        out_specs=[
            pl.BlockSpec((_RTB, 1), lambda i: (i, 0)),
            pl.BlockSpec((_RTB, 1), lambda i: (i, 0)),
            pl.BlockSpec((_RTB, 1), lambda i: (i, 0)),
            pl.BlockSpec((_RTB, 1), lambda i: (i, 0)),
            pl.BlockSpec((_RTB, K), lambda i: (i, 0)),
            pl.BlockSpec((_RTB, D2), lambda i: (i, 0)),
        ],
        out_shape=[
            jax.ShapeDtypeStruct((T, 1), jnp.int32),
            jax.ShapeDtypeStruct((T, 1), jnp.int32),
            jax.ShapeDtypeStruct((T, 1), jnp.int32),
            jax.ShapeDtypeStruct((T, 1), jnp.int32),
            jax.ShapeDtypeStruct((T, K), jnp.float32),
            jax.ShapeDtypeStruct((T, D2), jnp.int32),
        ],
        scratch_shapes=[pltpu.VMEM((1, E), jnp.float32)],
    )(x, w_router)


def _dispatch_body(x_hbm, si0_hbm, si1_hbm, buf_hbm,
                   rows_v, i0_v, i1_v, sem0, sem1):
    wid = lax.axis_index("s") * 2 + lax.axis_index("c")
    tb = wid * TPW
    pltpu.sync_copy(x_hbm.at[pl.ds(tb, TCHUNK)], rows_v)
    pltpu.sync_copy(si0_hbm.at[pl.ds(tb, TCHUNK)], i0_v)
    pltpu.sync_copy(si1_hbm.at[pl.ds(tb, TCHUNK)], i1_v)
    c0 = pltpu.async_copy(rows_v, buf_hbm.at[i0_v], sem0)
    c1 = pltpu.async_copy(rows_v, buf_hbm.at[i1_v], sem1)
    c0.wait()
    c1.wait()


@functools.cache
def _make_dispatch():
    return pl.kernel(
        _dispatch_body,
        out_type=jax.ShapeDtypeStruct(((E + 1) * CAP, D2), jnp.int32),
        mesh=plsc.VectorSubcoreMesh(core_axis_name="c", subcore_axis_name="s"),
        scratch_types=[
            pltpu.VMEM((TCHUNK, D2), jnp.int32),
            pltpu.VMEM((TCHUNK,), jnp.int32),
            pltpu.VMEM((TCHUNK,), jnp.int32),
            pltpu.SemaphoreType.DMA,
            pltpu.SemaphoreType.DMA,
        ],
    )


_NF = 2
_FB = F // _NF


def _mlp_body(buf_ref, wu_ref, wd_ref, y_ref, acc_ref):
    h = jnp.maximum(
        jnp.dot(_unpack_row(buf_ref[...]), wu_ref[0],
                preferred_element_type=jnp.float32),
        0.0)
    part = jnp.dot(h, wd_ref[0], preferred_element_type=jnp.float32)
    f = pl.program_id(1)

    @pl.when(f == 0)
    def _first():
        acc_ref[...] = part

    @pl.when(f == _NF - 1)
    def _last():
        y_ref[...] = _pack_row(acc_ref[...] + part)

    @pl.when((f > 0) & (f < _NF - 1))
    def _mid():
        acc_ref[...] += part


def _mlp(buf, w_up, w_down):
    return pl.pallas_call(
        _mlp_body,
        grid=(E, _NF),
        in_specs=[
            pl.BlockSpec((CAP, D2), lambda e, f: (e, 0)),
            pl.BlockSpec((1, D, _FB), lambda e, f: (e, 0, f)),
            pl.BlockSpec((1, _FB, D), lambda e, f: (e, f, 0)),
        ],
        out_specs=pl.BlockSpec((CAP, D2), lambda e, f: (e, 0)),
        out_shape=jax.ShapeDtypeStruct((E * CAP, D2), jnp.int32),
        scratch_shapes=[pltpu.VMEM((CAP, D), jnp.float32)],
    )(buf, w_up, w_down)


def _gather_body(y_hbm, gi0_hbm, gi1_hbm, r0_hbm, r1_hbm,
                 rows0_v, rows1_v, i0_v, i1_v, sem0, sem1):
    wid = lax.axis_index("s") * 2 + lax.axis_index("c")
    tb = wid * TPW
    pltpu.sync_copy(gi0_hbm.at[pl.ds(tb, TCHUNK)], i0_v)
    pltpu.sync_copy(gi1_hbm.at[pl.ds(tb, TCHUNK)], i1_v)
    g0 = pltpu.async_copy(y_hbm.at[i0_v], rows0_v, sem0)
    g1 = pltpu.async_copy(y_hbm.at[i1_v], rows1_v, sem1)
    g0.wait()
    o0 = pltpu.async_copy(rows0_v, r0_hbm.at[pl.ds(tb, TCHUNK)], sem0)
    g1.wait()
    o1 = pltpu.async_copy(rows1_v, r1_hbm.at[pl.ds(tb, TCHUNK)], sem1)
    o0.wait()
    o1.wait()


@functools.cache
def _make_gather_pairs():
    return pl.kernel(
        _gather_body,
        out_type=[
            jax.ShapeDtypeStruct((T, D2), jnp.int32),
            jax.ShapeDtypeStruct((T, D2), jnp.int32),
        ],
        mesh=plsc.VectorSubcoreMesh(core_axis_name="c", subcore_axis_name="s"),
        scratch_types=[
            pltpu.VMEM((TCHUNK, D2), jnp.int32),
            pltpu.VMEM((TCHUNK, D2), jnp.int32),
            pltpu.VMEM((TCHUNK,), jnp.int32),
            pltpu.VMEM((TCHUNK,), jnp.int32),
            pltpu.SemaphoreType.DMA,
            pltpu.SemaphoreType.DMA,
        ],
    )


_CTB = 512


def _combine_body(r0_ref, r1_ref, w_ref, out_ref):
    w0 = w_ref[:, 0:1]
    w1 = w_ref[:, 1:2]
    r0 = _unpack_row(r0_ref[...])
    r1 = _unpack_row(r1_ref[...])
    out_ref[...] = (jnp.where(w0 == 0.0, 0.0, w0 * r0)
                    + jnp.where(w1 == 0.0, 0.0, w1 * r1))


def _combine(r0, r1, w):
    return pl.pallas_call(
        _combine_body,
        grid=(T // _CTB,),
        in_specs=[
            pl.BlockSpec((_CTB, D2), lambda i: (i, 0)),
            pl.BlockSpec((_CTB, D2), lambda i: (i, 0)),
            pl.BlockSpec((_CTB, K), lambda i: (i, 0)),
        ],
        out_specs=pl.BlockSpec((_CTB, D), lambda i: (i, 0)),
        out_shape=jax.ShapeDtypeStruct((T, D), jnp.float32),
    )(r0, r1, w)


def kernel(hidden_states, w_router, w_up, w_down):
    si0, si1, gi0, gi1, w, xpk = _router(hidden_states, w_router)
    buf = _make_dispatch()(xpk, si0.reshape(-1), si1.reshape(-1))
    y = _mlp(buf, w_up, w_down)
    r0, r1 = _make_gather_pairs()(y, gi0.reshape(-1), gi1.reshape(-1))
    return _combine(r0, r1, w)


# speedup vs baseline: 1.2294x; 1.2294x over previous
import functools

import jax
import jax.numpy as jnp
from jax import lax
from jax.experimental import pallas as pl
from jax.experimental.pallas import tpu as pltpu
from jax.experimental.pallas import tpu_sc as plsc

T = 4096
D = 768
E = 64
K = 2
F = 1024
CAP = 256
TRASH = E * CAP
NW = 32
TPW = T // NW
TCHUNK = 128
D2 = D // 2


def _pack_row(v):
    bits = jax.lax.bitcast_convert_type(v, jnp.int32) + 0x8000
    hi16 = jax.lax.shift_right_logical(bits, 16)
    lo = hi16[:, :D2] & 0xFFFF
    hi = jax.lax.shift_left(hi16[:, D2:], 16)
    return lo | hi


def _unpack_row(w):
    a = jax.lax.bitcast_convert_type(jax.lax.shift_left(w, 16), jnp.float32)
    b = jax.lax.bitcast_convert_type(w & jnp.int32(-65536), jnp.float32)
    return jnp.concatenate([a, b], axis=1)


_RTB = 512


def _router_body(x_ref, wr_ref, si0_ref, si1_ref, gi0_ref, gi1_ref, w_ref,
                 xpk_ref, off_ref):
    @pl.when(pl.program_id(0) == 0)
    def _init():
        off_ref[...] = jnp.zeros((1, E), jnp.float32)

    x = x_ref[...]
    xpk_ref[...] = _pack_row(x)
    logits = jnp.dot(x, wr_ref[...], preferred_element_type=jnp.float32)
    lanes = lax.broadcasted_iota(jnp.int32, logits.shape, 1)
    m0 = jnp.max(logits, axis=-1, keepdims=True)
    i0 = jnp.min(jnp.where(logits == m0, lanes, E), axis=-1, keepdims=True)
    neg = jnp.float32(-jnp.inf)
    p1 = jnp.where(lanes == i0, neg, logits)
    m1 = jnp.max(p1, axis=-1, keepdims=True)
    i1 = jnp.min(jnp.where(p1 == m1, lanes, E), axis=-1, keepdims=True)
    oh0 = (lanes == i0).astype(jnp.float32)
    oh1 = (lanes == i1).astype(jnp.float32)
    h = oh0 + oh1
    _B = 256
    rr = lax.broadcasted_iota(jnp.int32, (_B, _B), 0)
    cc = lax.broadcasted_iota(jnp.int32, (_B, _B), 1)
    tri = (rr > cc).astype(jnp.float32)
    parts = []
    off = off_ref[...]
    for b in range(_RTB // _B):
        hb = h[b * _B:(b + 1) * _B, :]
        parts.append(
            jnp.dot(tri, hb, preferred_element_type=jnp.float32) + off)
        off = off + jnp.sum(hb, axis=0, keepdims=True)
    off_ref[...] = off
    cex = jnp.concatenate(parts, axis=0)
    pos0 = jnp.sum(cex * oh0, axis=-1, keepdims=True).astype(jnp.int32)
    pos1 = jnp.sum(cex * oh1, axis=-1, keepdims=True).astype(jnp.int32)
    q = jnp.exp(m1 - m0)
    w1 = q / (1.0 + q)
    w0 = 1.0 - w1
    v0 = pos0 < CAP
    v1 = pos1 < CAP
    slot0 = i0 * CAP + jnp.minimum(pos0, CAP - 1)
    slot1 = i1 * CAP + jnp.minimum(pos1, CAP - 1)
    si0_ref[...] = jnp.where(v0, slot0, TRASH)
    si1_ref[...] = jnp.where(v1, slot1, TRASH)
    gi0_ref[...] = slot0
    gi1_ref[...] = slot1
    w_ref[...] = jnp.concatenate(
        [jnp.where(v0, w0, 0.0), jnp.where(v1, w1, 0.0)], axis=1)


def _router(x, w_router):
    return pl.pallas_call(
        _router_body,
        grid=(T // _RTB,),
        in_specs=[
            pl.BlockSpec((_RTB, D), lambda i: (i, 0)),
            pl.BlockSpec((D, E), lambda i: (0, 0)),
        ],
        out_specs=[
            pl.BlockSpec((_RTB, 1), lambda i: (i, 0)),
            pl.BlockSpec((_RTB, 1), lambda i: (i, 0)),
            pl.BlockSpec((_RTB, 1), lambda i: (i, 0)),
            pl.BlockSpec((_RTB, 1), lambda i: (i, 0)),
            pl.BlockSpec((_RTB, K), lambda i: (i, 0)),
            pl.BlockSpec((_RTB, D2), lambda i: (i, 0)),
        ],
        out_shape=[
            jax.ShapeDtypeStruct((T, 1), jnp.int32),
            jax.ShapeDtypeStruct((T, 1), jnp.int32),
            jax.ShapeDtypeStruct((T, 1), jnp.int32),
            jax.ShapeDtypeStruct((T, 1), jnp.int32),
            jax.ShapeDtypeStruct((T, K), jnp.float32),
            jax.ShapeDtypeStruct((T, D2), jnp.int32),
        ],
        scratch_shapes=[pltpu.VMEM((1, E), jnp.float32)],
    )(x, w_router)


def _dispatch_body(x_hbm, si0_hbm, si1_hbm, buf_hbm,
                   rows_v, i0_v, i1_v, sem0, sem1):
    wid = lax.axis_index("s") * 2 + lax.axis_index("c")
    tb = wid * TPW
    pltpu.sync_copy(x_hbm.at[pl.ds(tb, TCHUNK)], rows_v)
    pltpu.sync_copy(si0_hbm.at[pl.ds(tb, TCHUNK)], i0_v)
    pltpu.sync_copy(si1_hbm.at[pl.ds(tb, TCHUNK)], i1_v)
    c0 = pltpu.async_copy(rows_v, buf_hbm.at[i0_v], sem0)
    c1 = pltpu.async_copy(rows_v, buf_hbm.at[i1_v], sem1)
    c0.wait()
    c1.wait()


@functools.cache
def _make_dispatch():
    return pl.kernel(
        _dispatch_body,
        out_type=jax.ShapeDtypeStruct(((E + 1) * CAP, D2), jnp.int32),
        mesh=plsc.VectorSubcoreMesh(core_axis_name="c", subcore_axis_name="s"),
        scratch_types=[
            pltpu.VMEM((TCHUNK, D2), jnp.int32),
            pltpu.VMEM((TCHUNK,), jnp.int32),
            pltpu.VMEM((TCHUNK,), jnp.int32),
            pltpu.SemaphoreType.DMA,
            pltpu.SemaphoreType.DMA,
        ],
    )


def _mlp_body(buf_ref, wu_ref, wd_ref, y_ref):
    h = jnp.maximum(
        jnp.dot(_unpack_row(buf_ref[...]), wu_ref[0],
                preferred_element_type=jnp.float32),
        0.0)
    y_ref[...] = _pack_row(
        jnp.dot(h, wd_ref[0], preferred_element_type=jnp.float32))


def _mlp(buf, w_up, w_down):
    return pl.pallas_call(
        _mlp_body,
        grid=(E,),
        in_specs=[
            pl.BlockSpec((CAP, D2), lambda e: (e, 0)),
            pl.BlockSpec((1, D, F), lambda e: (e, 0, 0)),
            pl.BlockSpec((1, F, D), lambda e: (e, 0, 0)),
        ],
        out_specs=pl.BlockSpec((CAP, D2), lambda e: (e, 0)),
        out_shape=jax.ShapeDtypeStruct((E * CAP, D2), jnp.int32),
    )(buf, w_up, w_down)


def _gather_body(y_hbm, gi0_hbm, gi1_hbm, r0_hbm, r1_hbm,
                 rows0_v, rows1_v, i0_v, i1_v, sem0, sem1):
    wid = lax.axis_index("s") * 2 + lax.axis_index("c")
    tb = wid * TPW
    pltpu.sync_copy(gi0_hbm.at[pl.ds(tb, TCHUNK)], i0_v)
    pltpu.sync_copy(gi1_hbm.at[pl.ds(tb, TCHUNK)], i1_v)
    g0 = pltpu.async_copy(y_hbm.at[i0_v], rows0_v, sem0)
    g1 = pltpu.async_copy(y_hbm.at[i1_v], rows1_v, sem1)
    g0.wait()
    o0 = pltpu.async_copy(rows0_v, r0_hbm.at[pl.ds(tb, TCHUNK)], sem0)
    g1.wait()
    o1 = pltpu.async_copy(rows1_v, r1_hbm.at[pl.ds(tb, TCHUNK)], sem1)
    o0.wait()
    o1.wait()


@functools.cache
def _make_gather_pairs():
    return pl.kernel(
        _gather_body,
        out_type=[
            jax.ShapeDtypeStruct((T, D2), jnp.int32),
            jax.ShapeDtypeStruct((T, D2), jnp.int32),
        ],
        mesh=plsc.VectorSubcoreMesh(core_axis_name="c", subcore_axis_name="s"),
        scratch_types=[
            pltpu.VMEM((TCHUNK, D2), jnp.int32),
            pltpu.VMEM((TCHUNK, D2), jnp.int32),
            pltpu.VMEM((TCHUNK,), jnp.int32),
            pltpu.VMEM((TCHUNK,), jnp.int32),
            pltpu.SemaphoreType.DMA,
            pltpu.SemaphoreType.DMA,
        ],
    )


_CTB = 512


def _combine_body(r0_ref, r1_ref, w_ref, out_ref):
    w0 = w_ref[:, 0:1]
    w1 = w_ref[:, 1:2]
    r0 = _unpack_row(r0_ref[...])
    r1 = _unpack_row(r1_ref[...])
    out_ref[...] = (jnp.where(w0 == 0.0, 0.0, w0 * r0)
                    + jnp.where(w1 == 0.0, 0.0, w1 * r1))


def _combine(r0, r1, w):
    return pl.pallas_call(
        _combine_body,
        grid=(T // _CTB,),
        in_specs=[
            pl.BlockSpec((_CTB, D2), lambda i: (i, 0)),
            pl.BlockSpec((_CTB, D2), lambda i: (i, 0)),
            pl.BlockSpec((_CTB, K), lambda i: (i, 0)),
        ],
        out_specs=pl.BlockSpec((_CTB, D), lambda i: (i, 0)),
        out_shape=jax.ShapeDtypeStruct((T, D), jnp.float32),
    )(r0, r1, w)


def kernel(hidden_states, w_router, w_up, w_down):
    si0, si1, gi0, gi1, w, xpk = _router(hidden_states, w_router)
    buf = _make_dispatch()(xpk, si0.reshape(-1), si1.reshape(-1))
    y = _mlp(buf, w_up, w_down)
    r0, r1 = _make_gather_pairs()(y, gi0.reshape(-1), gi1.reshape(-1))
    return _combine(r0, r1, w)
